# Initial kernel scaffold; baseline (speedup 1.0000x reference)
import jax
import jax.numpy as jnp
from jax.experimental import pallas as pl


def kernel(x_cat, x_num, emb, W1, b1, W2, b2, W3, b3, W4, b4):
    raise NotImplementedError("write your pallas kernel here")


# speedup vs baseline: 1.1288x; 1.1288x over previous
import jax
import jax.numpy as jnp
from jax import lax
from jax.experimental import pallas as pl
from jax.experimental.pallas import tpu as pltpu
from jax.experimental.pallas import tpu_sc as plsc

_F = 26
_V = 100000
_E = 32
_NC = 2
_NS = 16
_NW = _NC * _NS
_CH = 128


def _gather_body(emb_hbm, idx_hbm, out_hbm, idx_v, buf_v, sem):
    nch = idx_hbm.shape[1]
    wid = lax.axis_index("s") * _NC + lax.axis_index("c")
    base = wid * (nch * _CH)
    pltpu.sync_copy(idx_hbm.at[wid], idx_v)

    def step(j, carry):
        pltpu.async_copy(emb_hbm.at[idx_v.at[j]], buf_v, sem).wait()
        pltpu.sync_copy(buf_v, out_hbm.at[pl.ds(base + j * _CH, _CH)])
        return carry

    lax.fori_loop(0, nch, step, 0)


def _sc_gather(emb_flat, idx):
    nch = idx.shape[1]
    kern = pl.kernel(
        _gather_body,
        out_type=jax.ShapeDtypeStruct((_NW * nch * _CH, _E), jnp.float32),
        mesh=plsc.VectorSubcoreMesh(core_axis_name="c", subcore_axis_name="s"),
        scratch_types=[
            pltpu.VMEM((nch, _CH), jnp.int32),
            pltpu.VMEM((_CH, _E), jnp.float32),
            pltpu.SemaphoreType.DMA,
        ],
        compiler_params=pltpu.CompilerParams(use_tc_tiling_on_sc=False),
    )
    return kern(emb_flat, idx)


def _mlp_body(flat_ref, xnum_ref, w1a_ref, w1b_ref, b1_ref, w2_ref, b2_ref,
              w3_ref, b3_ref, w4_ref, b4_ref, out_ref):
    h = jnp.dot(flat_ref[...], w1a_ref[...], preferred_element_type=jnp.float32)
    h += jnp.dot(xnum_ref[...], w1b_ref[...], preferred_element_type=jnp.float32)
    h = jnp.maximum(h + b1_ref[...], 0.0)
    h = jnp.maximum(
        jnp.dot(h, w2_ref[...], preferred_element_type=jnp.float32) + b2_ref[...], 0.0)
    h = jnp.maximum(
        jnp.dot(h, w3_ref[...], preferred_element_type=jnp.float32) + b3_ref[...], 0.0)
    out_ref[...] = (
        jnp.dot(h, w4_ref[...], preferred_element_type=jnp.float32) + b4_ref[...])


def _tc_mlp(flat, xnum, w1a, w1b, b1, w2, b2, w3, b3, w4, b4):
    batch = flat.shape[0]
    bb = 1024
    full = lambda a: pl.BlockSpec(a.shape, lambda i: (0, 0))
    return pl.pallas_call(
        _mlp_body,
        grid=(batch // bb,),
        in_specs=[
            pl.BlockSpec((bb, flat.shape[1]), lambda i: (i, 0)),
            pl.BlockSpec((bb, xnum.shape[1]), lambda i: (i, 0)),
            full(w1a), full(w1b), full(b1), full(w2), full(b2),
            full(w3), full(b3), full(w4), full(b4),
        ],
        out_specs=pl.BlockSpec((bb, 1), lambda i: (i, 0)),
        out_shape=jax.ShapeDtypeStruct((batch, 1), jnp.float32),
    )(flat, xnum, w1a, w1b, b1, w2, b2, w3, b3, w4, b4)


def kernel(x_cat, x_num, emb, W1, b1, W2, b2, W3, b3, W4, b4):
    batch = x_cat.shape[0]
    emb_flat = emb.reshape(_F * _V, _E)
    offs = (jnp.arange(_F, dtype=jnp.int32) * _V)[None, :]
    idx = (x_cat.astype(jnp.int32) + offs).reshape(_NW, (batch * _F) // (_NW * _CH), _CH)
    flat = _sc_gather(emb_flat, idx).reshape(batch, _F * _E)

    xnum_p = jnp.pad(x_num, ((0, 0), (0, 3)))
    w1a = W1[:_F * _E]
    w1b = jnp.pad(W1[_F * _E:], ((0, 3), (0, 0)))
    return _tc_mlp(
        flat, xnum_p, w1a, w1b,
        b1.reshape(1, -1), W2, b2.reshape(1, -1),
        W3, b3.reshape(1, -1), W4, b4.reshape(1, -1))


# speedup vs baseline: 6.1980x; 5.4907x over previous
import jax
import jax.numpy as jnp
from jax import lax
from jax.experimental import pallas as pl
from jax.experimental.pallas import tpu as pltpu
from jax.experimental.pallas import tpu_sc as plsc

_F = 26
_V = 100000
_E = 32
_NC = 2
_NS = 16
_NW = _NC * _NS
_L = 16


def _gather_t_body(emb_ref, idx_ref, out_ref, row_v, idxrow_v, outrow_v):
    batch = idx_ref.shape[1]
    w = lax.axis_index("s") * _NC + lax.axis_index("c")

    def field_step(k, carry):
        r = _E * k + w
        pltpu.sync_copy(idx_ref.at[k], idxrow_v)
        pltpu.sync_copy(emb_ref.at[r], row_v)

        def g(i, c):
            iv = idxrow_v[pl.ds(_L * i, _L)]
            outrow_v[pl.ds(_L * i, _L)] = plsc.load_gather(row_v, [iv])
            return c

        lax.fori_loop(0, batch // _L, g, 0)
        pltpu.sync_copy(outrow_v, out_ref.at[r])
        return carry

    lax.fori_loop(0, _F, field_step, 0)


def _sc_gather_t(emb_t, idx_t):
    batch = idx_t.shape[1]
    kern = pl.kernel(
        _gather_t_body,
        out_type=jax.ShapeDtypeStruct((_F * _E, batch), jnp.float32),
        mesh=plsc.VectorSubcoreMesh(core_axis_name="c", subcore_axis_name="s"),
        scratch_types=[
            pltpu.VMEM((_V,), jnp.float32),
            pltpu.VMEM((batch,), jnp.int32),
            pltpu.VMEM((batch,), jnp.float32),
        ],
        compiler_params=pltpu.CompilerParams(needs_layout_passes=False),
    )
    return kern(emb_t, idx_t)


def _dot0(a, b):
    return lax.dot_general(a, b, (((0,), (0,)), ((), ())),
                           preferred_element_type=jnp.float32)


def _mlp_body(x_ref, xnum_ref, w1a_ref, w1b_ref, b1_ref, w2_ref, b2_ref,
              w3_ref, b3_ref, w4_ref, b4_ref, out_ref):
    h = _dot0(w1a_ref[...], x_ref[...])
    h += _dot0(w1b_ref[...], xnum_ref[...])
    h = jnp.maximum(h + b1_ref[...], 0.0)
    h = jnp.maximum(_dot0(w2_ref[...], h) + b2_ref[...], 0.0)
    h = jnp.maximum(_dot0(w3_ref[...], h) + b3_ref[...], 0.0)
    out_ref[...] = _dot0(w4_ref[...], h) + b4_ref[...]


def _tc_mlp_t(x_t, xnum_t, w1a, w1b, b1, w2, b2, w3, b3, w4, b4):
    batch = x_t.shape[1]
    bb = 1024
    full = lambda a: pl.BlockSpec(a.shape, lambda i: (0, 0))
    return pl.pallas_call(
        _mlp_body,
        grid=(batch // bb,),
        in_specs=[
            pl.BlockSpec((x_t.shape[0], bb), lambda i: (0, i)),
            pl.BlockSpec((xnum_t.shape[0], bb), lambda i: (0, i)),
            full(w1a), full(w1b), full(b1), full(w2), full(b2),
            full(w3), full(b3), full(w4), full(b4),
        ],
        out_specs=pl.BlockSpec((1, bb), lambda i: (0, i)),
        out_shape=jax.ShapeDtypeStruct((1, batch), jnp.float32),
    )(x_t, xnum_t, w1a, w1b, b1, w2, b2, w3, b3, w4, b4)


def kernel(x_cat, x_num, emb, W1, b1, W2, b2, W3, b3, W4, b4):
    batch = x_cat.shape[0]
    emb_t = jnp.transpose(emb, (0, 2, 1)).reshape(_F * _E, _V)
    idx_t = jnp.transpose(x_cat.astype(jnp.int32))
    x_t = _sc_gather_t(emb_t, idx_t)

    xnum_t = jnp.pad(jnp.transpose(x_num), ((0, 3), (0, 0)))
    w1a = W1[:_F * _E]
    w1b = jnp.pad(W1[_F * _E:], ((0, 3), (0, 0)))
    out_t = _tc_mlp_t(
        x_t, xnum_t, w1a, w1b,
        b1.reshape(-1, 1), W2, b2.reshape(-1, 1),
        W3, b3.reshape(-1, 1), W4, b4.reshape(-1, 1))
    return out_t.reshape(batch, 1)


# speedup vs baseline: 6.2111x; 1.0021x over previous
import jax
import jax.numpy as jnp
from jax import lax
from jax.experimental import pallas as pl
from jax.experimental.pallas import tpu as pltpu
from jax.experimental.pallas import tpu_sc as plsc

_F = 26
_V = 100000
_E = 32
_NC = 2
_NS = 16
_NW = _NC * _NS
_L = 16


def _gather_t_body(emb_ref, idx_ref, out_ref, row_v, idxrow_v, outrow_v):
    batch = idx_ref.shape[1]
    w = lax.axis_index("s") * _NC + lax.axis_index("c")

    def field_step(k, carry):
        r = _E * k + w
        pltpu.sync_copy(idx_ref.at[k], idxrow_v)
        pltpu.sync_copy(emb_ref.at[r], row_v)

        def g(i, c):
            iv = idxrow_v[pl.ds(_L * i, _L)]
            outrow_v[pl.ds(_L * i, _L)] = plsc.load_gather(row_v, [iv])
            return c

        lax.fori_loop(0, batch // _L, g, 0)
        pltpu.sync_copy(outrow_v, out_ref.at[r])
        return carry

    lax.fori_loop(0, _F, field_step, 0)


def _sc_gather_t(emb_t, idx_t):
    batch = idx_t.shape[1]
    kern = pl.kernel(
        _gather_t_body,
        out_type=jax.ShapeDtypeStruct((_F * _E, batch), jnp.float32),
        mesh=plsc.VectorSubcoreMesh(core_axis_name="c", subcore_axis_name="s"),
        scratch_types=[
            pltpu.VMEM((_V,), jnp.float32),
            pltpu.VMEM((batch,), jnp.int32),
            pltpu.VMEM((batch,), jnp.float32),
        ],
        compiler_params=pltpu.CompilerParams(needs_layout_passes=False),
    )
    return kern(emb_t, idx_t)


def _dot0(a, b):
    return lax.dot_general(a, b, (((0,), (0,)), ((), ())),
                           preferred_element_type=jnp.float32)


def _mlp_body(x_ref, xnum_ref, w1a_ref, w1b_ref, b1_ref, w2_ref, b2_ref,
              w3_ref, b3_ref, w4_ref, b4_ref, out_ref):
    h = _dot0(w1a_ref[...], x_ref[...])
    h += _dot0(w1b_ref[...], xnum_ref[...])
    h = jnp.maximum(h + b1_ref[...], 0.0)
    h = jnp.maximum(_dot0(w2_ref[...], h) + b2_ref[...], 0.0)
    h = jnp.maximum(_dot0(w3_ref[...], h) + b3_ref[...], 0.0)
    out_ref[...] = _dot0(w4_ref[...], h) + b4_ref[...]


def _tc_mlp_t(x_t, xnum_t, w1a, w1b, b1, w2, b2, w3, b3, w4, b4):
    batch = x_t.shape[1]
    bb = 2048
    full = lambda a: pl.BlockSpec(a.shape, lambda i: (0, 0))
    return pl.pallas_call(
        _mlp_body,
        grid=(batch // bb,),
        in_specs=[
            pl.BlockSpec((x_t.shape[0], bb), lambda i: (0, i)),
            pl.BlockSpec((xnum_t.shape[0], bb), lambda i: (0, i)),
            full(w1a), full(w1b), full(b1), full(w2), full(b2),
            full(w3), full(b3), full(w4), full(b4),
        ],
        out_specs=pl.BlockSpec((1, bb), lambda i: (0, i)),
        out_shape=jax.ShapeDtypeStruct((1, batch), jnp.float32),
    )(x_t, xnum_t, w1a, w1b, b1, w2, b2, w3, b3, w4, b4)


def kernel(x_cat, x_num, emb, W1, b1, W2, b2, W3, b3, W4, b4):
    batch = x_cat.shape[0]
    emb_t = jnp.transpose(emb, (0, 2, 1)).reshape(_F * _E, _V)
    idx_t = jnp.transpose(x_cat.astype(jnp.int32))
    x_t = _sc_gather_t(emb_t, idx_t)

    xnum_t = jnp.pad(jnp.transpose(x_num), ((0, 3), (0, 0)))
    w1a = W1[:_F * _E]
    w1b = jnp.pad(W1[_F * _E:], ((0, 3), (0, 0)))
    out_t = _tc_mlp_t(
        x_t, xnum_t, w1a, w1b,
        b1.reshape(-1, 1), W2, b2.reshape(-1, 1),
        W3, b3.reshape(-1, 1), W4, b4.reshape(-1, 1))
    return out_t.reshape(batch, 1)


# speedup vs baseline: 6.7782x; 1.0913x over previous
import jax
import jax.numpy as jnp
from jax import lax
from jax.experimental import pallas as pl
from jax.experimental.pallas import tpu as pltpu
from jax.experimental.pallas import tpu_sc as plsc

_F = 26
_V = 100000
_E = 32
_NC = 2
_NS = 16
_NW = _NC * _NS
_L = 16
_RPW = (_F * _E) // _NW


def _rne_bf16_hi(u):
    bit = lax.shift_right_logical(u, 16) & jnp.int32(1)
    return lax.shift_right_logical(u + jnp.int32(0x7FFF) + bit, 16)


def _gather_t_body(emb_ref, idx_ref, out_ref, row_v, idxrow_v, outw_v):
    batch = idx_ref.shape[1]
    w = lax.axis_index("s") * _NC + lax.axis_index("c")
    start = _RPW * w
    lanes = lax.iota(jnp.int32, _L)

    def per_field(f, carry):
        pltpu.sync_copy(idx_ref.at[f], idxrow_v)
        r0 = lax.max(start, _E * f)
        r1 = lax.min(start + _RPW, _E * (f + 1))

        def per_row(r, c2):
            pltpu.sync_copy(emb_ref.at[r], row_v)

            def g(i, c3):
                pe = _L * 2 * i + 2 * lanes
                ie = plsc.load_gather(idxrow_v, [pe])
                io = plsc.load_gather(idxrow_v, [pe + 1])
                a = plsc.load_gather(row_v, [ie])
                b = plsc.load_gather(row_v, [io])
                ra = _rne_bf16_hi(plsc.bitcast(a, jnp.int32))
                rb = _rne_bf16_hi(plsc.bitcast(b, jnp.int32))
                outw_v[pl.ds(_L * i, _L)] = ra | lax.shift_left(rb, 16)
                return c3

            lax.fori_loop(0, batch // (2 * _L), g, 0)
            pltpu.sync_copy(outw_v, out_ref.at[r])
            return c2

        return lax.fori_loop(r0, r1, per_row, carry)

    lax.fori_loop(start // _E, (start + _RPW - 1) // _E + 1, per_field, 0)


def _sc_gather_t(emb_t, idx_t):
    batch = idx_t.shape[1]
    kern = pl.kernel(
        _gather_t_body,
        out_type=jax.ShapeDtypeStruct((_F * _E, batch // 2), jnp.int32),
        mesh=plsc.VectorSubcoreMesh(core_axis_name="c", subcore_axis_name="s"),
        scratch_types=[
            pltpu.VMEM((_V,), jnp.float32),
            pltpu.VMEM((batch,), jnp.int32),
            pltpu.VMEM((batch // 2,), jnp.int32),
        ],
        compiler_params=pltpu.CompilerParams(needs_layout_passes=False),
    )
    return kern(emb_t, idx_t)


def _dot0(a, b):
    return lax.dot_general(a, b, (((0,), (0,)), ((), ())),
                           preferred_element_type=jnp.float32)


def _mlp_body(x_ref, xnum_ref, w1a_ref, w1b_ref, b1_ref, w2_ref, b2_ref,
              w3_ref, b3_ref, w4_ref, b4_ref, out_ref):
    x32 = x_ref[...]
    lo = lax.bitcast_convert_type(lax.shift_left(x32, 16), jnp.float32)
    hi = lax.bitcast_convert_type(x32 & jnp.int32(-65536), jnp.float32)
    xbf = jnp.concatenate([lo, hi], axis=1).astype(jnp.bfloat16)
    h = _dot0(w1a_ref[...].astype(jnp.bfloat16), xbf)
    h += _dot0(w1b_ref[...], xnum_ref[...])
    h = jnp.maximum(h + b1_ref[...], 0.0)
    h = jnp.maximum(_dot0(w2_ref[...], h) + b2_ref[...], 0.0)
    h = jnp.maximum(_dot0(w3_ref[...], h) + b3_ref[...], 0.0)
    out_ref[...] = _dot0(w4_ref[...], h) + b4_ref[...]


def _tc_mlp_t(xw, xnum_t, w1a, w1b, b1, w2, b2, w3, b3, w4, b4):
    nw = xw.shape[1]
    bbw = nw // 2
    bb = 2 * bbw
    full = lambda a: pl.BlockSpec(a.shape, lambda i: (0, 0))
    return pl.pallas_call(
        _mlp_body,
        grid=(nw // bbw,),
        in_specs=[
            pl.BlockSpec((xw.shape[0], bbw), lambda i: (0, i)),
            pl.BlockSpec((xnum_t.shape[0], bb), lambda i: (0, i)),
            full(w1a), full(w1b), full(b1), full(w2), full(b2),
            full(w3), full(b3), full(w4), full(b4),
        ],
        out_specs=pl.BlockSpec((1, bb), lambda i: (0, i)),
        out_shape=jax.ShapeDtypeStruct((1, 2 * nw), jnp.float32),
    )(xw, xnum_t, w1a, w1b, b1, w2, b2, w3, b3, w4, b4)


def kernel(x_cat, x_num, emb, W1, b1, W2, b2, W3, b3, W4, b4):
    batch = x_cat.shape[0]
    emb_t = jnp.transpose(emb, (0, 2, 1)).reshape(_F * _E, _V)
    idx_t = jnp.transpose(x_cat.astype(jnp.int32))
    xw = _sc_gather_t(emb_t, idx_t)

    nb = 2
    jcols = batch // (2 * nb)
    xnum_t = jnp.pad(jnp.transpose(x_num), ((0, 3), (0, 0)))
    xnum_p = (xnum_t.reshape(-1, nb, jcols, 2)
              .transpose(0, 1, 3, 2).reshape(-1, batch))
    w1a = W1[:_F * _E]
    w1b = jnp.pad(W1[_F * _E:], ((0, 3), (0, 0)))
    out_t = _tc_mlp_t(
        xw, xnum_p, w1a, w1b,
        b1.reshape(-1, 1), W2, b2.reshape(-1, 1),
        W3, b3.reshape(-1, 1), W4, b4.reshape(-1, 1))
    return (out_t.reshape(nb, 2, jcols).transpose(0, 2, 1)
            .reshape(batch, 1))


# speedup vs baseline: 8.1632x; 1.2043x over previous
import jax
import jax.numpy as jnp
from jax import lax
from jax.experimental import pallas as pl
from jax.experimental.pallas import tpu as pltpu
from jax.experimental.pallas import tpu_sc as plsc

_F = 26
_V = 100000
_E = 32
_NC = 2
_NS = 16
_NW = _NC * _NS
_L = 16
_RPW = (_F * _E) // _NW


def _rne_bf16_hi(u):
    bit = lax.shift_right_logical(u, 16) & jnp.int32(1)
    return lax.shift_right_logical(u + jnp.int32(0x7FFF) + bit, 16)


_VA = 49920
_VB = _V - _VA


def _gather_t_body(emb_ref, idx_ref, out_ref, bufa_v, bufb_v, idxrow_v,
                   outw_v, sema, semb):
    batch = idx_ref.shape[1]
    w = lax.axis_index("s") * _NC + lax.axis_index("c")
    start = _RPW * w
    end = start + _RPW
    lanes = lax.iota(jnp.int32, _L)

    def start_a(r):
        return pltpu.async_copy(emb_ref.at[r, pl.ds(0, _VA)], bufa_v, sema)

    def start_b(r):
        return pltpu.async_copy(emb_ref.at[r, pl.ds(_VA, _VB)], bufb_v, semb)

    start_a(start)
    start_b(start)

    def per_row(r, fprev):
        f = r // _E

        @pl.when(f != fprev)
        def _():
            pltpu.sync_copy(idx_ref.at[f], idxrow_v)

        pltpu.make_async_copy(emb_ref.at[r, pl.ds(0, _VA)], bufa_v, sema).wait()

        def g1(i, c):
            pe = _L * 2 * i + 2 * lanes
            ie = plsc.load_gather(idxrow_v, [pe])
            io = plsc.load_gather(idxrow_v, [pe + 1])
            a = plsc.load_gather(bufa_v, [jnp.minimum(ie, _VA - 1)])
            b = plsc.load_gather(bufa_v, [jnp.minimum(io, _VA - 1)])
            ra = _rne_bf16_hi(plsc.bitcast(a, jnp.int32))
            rb = _rne_bf16_hi(plsc.bitcast(b, jnp.int32))
            outw_v[pl.ds(_L * i, _L)] = ra | lax.shift_left(rb, 16)
            return c

        lax.fori_loop(0, batch // (2 * _L), g1, 0)

        @pl.when(r + 1 < end)
        def _():
            start_a(r + 1)

        pltpu.make_async_copy(emb_ref.at[r, pl.ds(_VA, _VB)], bufb_v, semb).wait()

        def g2(i, c):
            pe = _L * 2 * i + 2 * lanes
            ie = plsc.load_gather(idxrow_v, [pe])
            io = plsc.load_gather(idxrow_v, [pe + 1])
            a = plsc.load_gather(bufb_v, [jnp.maximum(ie - _VA, 0)])
            b = plsc.load_gather(bufb_v, [jnp.maximum(io - _VA, 0)])
            ra = _rne_bf16_hi(plsc.bitcast(a, jnp.int32))
            rb = _rne_bf16_hi(plsc.bitcast(b, jnp.int32))
            word = outw_v[pl.ds(_L * i, _L)]
            we = jnp.where(ie >= _VA, ra, word & jnp.int32(0xFFFF))
            wo = jnp.where(io >= _VA, rb, lax.shift_right_logical(word, 16))
            outw_v[pl.ds(_L * i, _L)] = we | lax.shift_left(wo, 16)
            return c

        lax.fori_loop(0, batch // (2 * _L), g2, 0)

        @pl.when(r + 1 < end)
        def _():
            start_b(r + 1)

        pltpu.sync_copy(outw_v, out_ref.at[r])
        return f

    lax.fori_loop(start, end, per_row, jnp.int32(-1))


def _sc_gather_t(emb_t, idx_t):
    batch = idx_t.shape[1]
    kern = pl.kernel(
        _gather_t_body,
        out_type=jax.ShapeDtypeStruct((_F * _E, batch // 2), jnp.int32),
        mesh=plsc.VectorSubcoreMesh(core_axis_name="c", subcore_axis_name="s"),
        scratch_types=[
            pltpu.VMEM((_VA,), jnp.float32),
            pltpu.VMEM((_VB,), jnp.float32),
            pltpu.VMEM((batch,), jnp.int32),
            pltpu.VMEM((batch // 2,), jnp.int32),
            pltpu.SemaphoreType.DMA,
            pltpu.SemaphoreType.DMA,
        ],
        compiler_params=pltpu.CompilerParams(needs_layout_passes=False),
    )
    return kern(emb_t, idx_t)


def _dot0(a, b):
    return lax.dot_general(a, b, (((0,), (0,)), ((), ())),
                           preferred_element_type=jnp.float32)


def _mlp_body(x_ref, xnum_ref, w1a_ref, w1b_ref, b1_ref, w2_ref, b2_ref,
              w3_ref, b3_ref, w4_ref, b4_ref, out_ref):
    x32 = x_ref[...]
    lo = lax.bitcast_convert_type(lax.shift_left(x32, 16), jnp.float32)
    hi = lax.bitcast_convert_type(x32 & jnp.int32(-65536), jnp.float32)
    xbf = jnp.concatenate([lo, hi], axis=1).astype(jnp.bfloat16)
    h = _dot0(w1a_ref[...].astype(jnp.bfloat16), xbf)
    h += _dot0(w1b_ref[...], xnum_ref[...])
    h = jnp.maximum(h + b1_ref[...], 0.0)
    h = jnp.maximum(_dot0(w2_ref[...], h) + b2_ref[...], 0.0)
    h = jnp.maximum(_dot0(w3_ref[...], h) + b3_ref[...], 0.0)
    out_ref[...] = _dot0(w4_ref[...], h) + b4_ref[...]


def _tc_mlp_t(xw, xnum_t, w1a, w1b, b1, w2, b2, w3, b3, w4, b4):
    nw = xw.shape[1]
    bbw = nw // 2
    bb = 2 * bbw
    full = lambda a: pl.BlockSpec(a.shape, lambda i: (0, 0))
    return pl.pallas_call(
        _mlp_body,
        grid=(nw // bbw,),
        in_specs=[
            pl.BlockSpec((xw.shape[0], bbw), lambda i: (0, i)),
            pl.BlockSpec((xnum_t.shape[0], bb), lambda i: (0, i)),
            full(w1a), full(w1b), full(b1), full(w2), full(b2),
            full(w3), full(b3), full(w4), full(b4),
        ],
        out_specs=pl.BlockSpec((1, bb), lambda i: (0, i)),
        out_shape=jax.ShapeDtypeStruct((1, 2 * nw), jnp.float32),
    )(xw, xnum_t, w1a, w1b, b1, w2, b2, w3, b3, w4, b4)


def kernel(x_cat, x_num, emb, W1, b1, W2, b2, W3, b3, W4, b4):
    batch = x_cat.shape[0]
    emb_t = jnp.transpose(emb, (0, 2, 1)).reshape(_F * _E, _V)
    idx_t = jnp.transpose(x_cat.astype(jnp.int32))
    xw = _sc_gather_t(emb_t, idx_t)

    nb = 2
    jcols = batch // (2 * nb)
    xnum_t = jnp.pad(jnp.transpose(x_num), ((0, 3), (0, 0)))
    xnum_p = (xnum_t.reshape(-1, nb, jcols, 2)
              .transpose(0, 1, 3, 2).reshape(-1, batch))
    w1a = W1[:_F * _E]
    w1b = jnp.pad(W1[_F * _E:], ((0, 3), (0, 0)))
    out_t = _tc_mlp_t(
        xw, xnum_p, w1a, w1b,
        b1.reshape(-1, 1), W2, b2.reshape(-1, 1),
        W3, b3.reshape(-1, 1), W4, b4.reshape(-1, 1))
    return (out_t.reshape(nb, 2, jcols).transpose(0, 2, 1)
            .reshape(batch, 1))
